# Initial kernel scaffold; baseline (speedup 1.0000x reference)
import jax
import jax.numpy as jnp
from jax.experimental import pallas as pl


def kernel(node_feat, edge_index, deg, W_msg0, b_msg0, W_upd0, b_upd0, W_msg1, b_msg1, W_upd1, b_upd1, W_ro, b_ro):
    raise NotImplementedError("write your pallas kernel here")


# speedup vs baseline: 6.0554x; 6.0554x over previous
import functools

import jax
import jax.numpy as jnp
from jax import lax
from jax.experimental import pallas as pl
from jax.experimental.pallas import tpu as pltpu
from jax.experimental.pallas import tpu_sc as plsc

NC = 2
NS = 16
BATCH = 256
CB = 8
DEG_SCALE = 16.0
U_MAX = 1.0


def _round_up(x, m):
    return (x + m - 1) // m * m


def _make_sc_segsum(width, n_pad, e_pad):
    nph = n_pad // 2
    acc_rows = _round_up(nph + 8, NS * BATCH)
    ept = e_pad // NS
    nb = ept // BATCH
    cb = CB
    nch = nb // cb
    RING = 6
    LAG = 3

    mesh = plsc.VectorSubcoreMesh(core_axis_name="c", subcore_axis_name="s")

    @functools.partial(
        pl.kernel,
        out_type=jax.ShapeDtypeStruct((n_pad, width), jnp.float32),
        mesh=mesh,
        scratch_types=[
            pltpu.VMEM((cb, BATCH), jnp.int32),
            pltpu.VMEM((cb, BATCH), jnp.int32),
            [pltpu.VMEM((BATCH, width), jnp.float32) for _ in range(RING)],
            pltpu.VMEM_SHARED((acc_rows, width), jnp.float32),
            [pltpu.SemaphoreType.DMA for _ in range(RING)],
            [pltpu.SemaphoreType.DMA for _ in range(RING)],
        ],
        compiler_params=pltpu.CompilerParams(use_tc_tiling_on_sc=False),
    )
    def segsum(table, src3, dstl4, zeros, out, src_v, dst_v, rows, acc,
               gsem, ssem):
        c = lax.axis_index("c")
        s = lax.axis_index("s")

        zrows = acc_rows // NS
        zbase = s * zrows

        def zero_body(j, carry):
            pltpu.sync_copy(zeros, acc.at[pl.ds(zbase + j * BATCH, BATCH)])
            return carry

        lax.fori_loop(0, zrows // BATCH, zero_body, 0)
        plsc.subcore_barrier()

        def chunk_body(ch, carry):
            pltpu.sync_copy(src3.at[s, pl.ds(ch * cb, cb)], src_v)
            pltpu.sync_copy(dstl4.at[c, s, pl.ds(ch * cb, cb)], dst_v)
            gd = {}
            sd = {}
            for b in range(cb + LAG):
                if b < cb:
                    if b >= RING:
                        sd[b - RING].wait()
                    gd[b] = pltpu.async_copy(
                        table.at[src_v.at[b]], rows[b % RING], gsem[b % RING])
                t = b - LAG
                if t >= 0 and t < cb:
                    gd[t].wait()
                    sd[t] = pltpu.async_copy(
                        rows[t % RING], acc.at[dst_v.at[t]], ssem[t % RING],
                        add=True)
            for t in range(max(0, cb - RING), cb):
                sd[t].wait()
            return carry

        lax.fori_loop(0, nch, chunk_body, 0)
        plsc.subcore_barrier()

        dr = nph // NS
        pltpu.sync_copy(acc.at[pl.ds(s * dr, dr)],
                        out.at[pl.ds(c * nph + s * dr, dr)])

    return segsum


def _bf(x):
    return x.astype(jnp.bfloat16).astype(jnp.float32)


def _dot(a, b):
    return jnp.dot(a, b, preferred_element_type=jnp.float32)


def _hdot(a, b):
    return jnp.dot(a, b, preferred_element_type=jnp.float32,
                   precision=jax.lax.Precision.HIGHEST)


def _tc_layer0(x0p, ssc0, Ab, Bb, bm, Wub, bu, blk):
    n_pad = x0p.shape[0]
    H = Wub.shape[1]
    Hh = H // 2

    def body(x_ref, s_ref, a_ref, bw_ref, bm_ref, u_ref, bu_ref, oa_ref,
             ob_ref):
        x = x_ref[...]
        sc = s_ref[...]
        t1 = _dot(x, a_ref[...]) + bm_ref[...]
        t2 = _hdot(sc, bw_ref[...])
        cnt = sc[:, 5:6]
        aggr = _bf(cnt * t1 + t2)
        cat = jnp.concatenate([x, aggr], axis=1)
        z = jnp.maximum(_dot(cat, u_ref[...]) + bu_ref[...], 0.0)
        zb = _bf(z)
        oa_ref[...] = zb[:, :Hh]
        ob_ref[...] = zb[:, Hh:]

    return pl.pallas_call(
        body,
        grid=(n_pad // blk,),
        in_specs=[
            pl.BlockSpec((blk, 8), lambda i: (i, 0)),
            pl.BlockSpec((blk, 8), lambda i: (i, 0)),
            pl.BlockSpec((8, H), lambda i: (0, 0)),
            pl.BlockSpec((8, H), lambda i: (0, 0)),
            pl.BlockSpec((1, H), lambda i: (0, 0)),
            pl.BlockSpec((8 + H, H), lambda i: (0, 0)),
            pl.BlockSpec((1, H), lambda i: (0, 0)),
        ],
        out_specs=[pl.BlockSpec((blk, Hh), lambda i: (i, 0)),
                   pl.BlockSpec((blk, Hh), lambda i: (i, 0))],
        out_shape=[jax.ShapeDtypeStruct((n_pad, Hh), jnp.float32),
                   jax.ShapeDtypeStruct((n_pad, Hh), jnp.float32)],
    )(x0p, ssc0, Ab, Bb, bm, Wub, bu)


def _tc_layer1(xa, xb, sa, sb, cnt2, Ab, Bb, bm, Wub, bu, wrob, bro, blk):
    n_pad = xa.shape[0]
    H = Ab.shape[0]
    Hh = H // 2

    def body(xa_ref, xb_ref, sa_ref, sb_ref, c_ref, a_ref, bw_ref, bm_ref,
             u_ref, bu_ref, w_ref, br_ref, o_ref):
        x = jnp.concatenate([xa_ref[...], xb_ref[...]], axis=1)
        s = jnp.concatenate([sa_ref[...], sb_ref[...]], axis=1)
        t1 = _dot(x, a_ref[...]) + bm_ref[...]
        t2 = _hdot(s, bw_ref[...])
        aggr = _bf(c_ref[...] * t1 + t2)
        cat = jnp.concatenate([x, aggr], axis=1)
        z = jnp.maximum(_dot(cat, u_ref[...]) + bu_ref[...], 0.0)
        r = _dot(_bf(z), w_ref[...])
        o_ref[...] = U_MAX * jnp.tanh(r + br_ref[...])

    return pl.pallas_call(
        body,
        grid=(n_pad // blk,),
        in_specs=[
            pl.BlockSpec((blk, Hh), lambda i: (i, 0)),
            pl.BlockSpec((blk, Hh), lambda i: (i, 0)),
            pl.BlockSpec((blk, Hh), lambda i: (i, 0)),
            pl.BlockSpec((blk, Hh), lambda i: (i, 0)),
            pl.BlockSpec((blk, 1), lambda i: (i, 0)),
            pl.BlockSpec((H, H), lambda i: (0, 0)),
            pl.BlockSpec((H, H), lambda i: (0, 0)),
            pl.BlockSpec((1, H), lambda i: (0, 0)),
            pl.BlockSpec((2 * H, H), lambda i: (0, 0)),
            pl.BlockSpec((1, H), lambda i: (0, 0)),
            pl.BlockSpec((H, 1), lambda i: (0, 0)),
            pl.BlockSpec((1, 1), lambda i: (0, 0)),
        ],
        out_specs=pl.BlockSpec((blk, 1), lambda i: (i, 0)),
        out_shape=jax.ShapeDtypeStruct((n_pad, 1), jnp.float32),
    )(xa, xb, sa, sb, cnt2, Ab, Bb, bm, Wub, bu, wrob, bro)


def kernel(node_feat, edge_index, deg, W_msg0, b_msg0, W_upd0, b_upd0,
           W_msg1, b_msg1, W_upd1, b_upd1, W_ro, b_ro):
    N = node_feat.shape[0]
    E = edge_index.shape[1]
    IN = node_feat.shape[1]
    H = W_msg0.shape[1]

    n_pad = _round_up(N, NC * NS * BATCH // 2)
    nph = n_pad // 2
    dummy = nph
    e_pad = _round_up(E, NS * BATCH * CB)

    src = edge_index[0].astype(jnp.int32)
    dst = edge_index[1].astype(jnp.int32)
    pad_e = e_pad - E
    src_p = jnp.pad(src, (0, pad_e))
    dmy = dummy + (jnp.arange(e_pad, dtype=jnp.int32) & 1023)
    d0 = jnp.where(dst < nph, dst, dmy[:E])
    d1 = jnp.where(dst >= nph, dst - nph, dmy[:E])
    dstl = jnp.stack([jnp.concatenate([d0, dmy[E:]]),
                      jnp.concatenate([d1, dmy[E:]])])
    nb = e_pad // NS // BATCH
    src3 = src_p.reshape(NS, nb, BATCH)
    dstl4 = dstl.reshape(NC, NS, nb, BATCH)

    x0 = jnp.concatenate([node_feat[:, :4], (deg / DEG_SCALE)[:, None]],
                         axis=1)
    x0p = jnp.zeros((n_pad, 8), jnp.float32)
    x0p = x0p.at[:N, :IN].set(_bf(x0))
    x0p = x0p.at[:N, 5].set(1.0)

    Ab0 = jnp.zeros((8, H), jnp.float32).at[:IN].set(_bf(W_msg0[:IN]))
    Bb0 = jnp.zeros((8, H), jnp.float32).at[:IN].set(_bf(W_msg0[IN:]))
    bm0 = b_msg0[None, :]
    Wu0p = jnp.zeros((8 + H, H), jnp.float32)
    Wu0p = Wu0p.at[:IN].set(_bf(W_upd0[:IN])).at[8:].set(_bf(W_upd0[IN:]))
    bu0 = b_upd0[None, :]

    Ab1 = _bf(W_msg1[:H])
    Bb1 = _bf(W_msg1[H:])
    bm1 = b_msg1[None, :]
    Wu1b = _bf(W_upd1)
    bu1 = b_upd1[None, :]
    wrob = _bf(W_ro)
    bro = b_ro[None, :]

    Hh = H // 2
    zeros8 = jnp.zeros((BATCH, 8), jnp.float32)
    zerosh = jnp.zeros((BATCH, Hh), jnp.float32)

    ssc0 = _make_sc_segsum(8, n_pad, e_pad)(x0p, src3, dstl4, zeros8)
    cnt2 = ssc0[:, 5:6]

    blk = 2048
    xa, xb = _tc_layer0(x0p, ssc0, Ab0, Bb0, bm0, Wu0p, bu0, blk)

    segh = _make_sc_segsum(Hh, n_pad, e_pad)
    sa = segh(xa, src3, dstl4, zerosh)
    sb = segh(xb, src3, dstl4, zerosh)

    out = _tc_layer1(xa, xb, sa, sb, cnt2, Ab1, Bb1, bm1, Wu1b, bu1,
                     wrob, bro, blk)
    return out[:N, 0]
